# baseline (speedup 1.0000x reference)
import functools
import math

import jax
import jax.numpy as jnp
from jax import lax
from jax.experimental import pallas as pl
from jax.experimental.pallas import tpu as pltpu
from jax.experimental.pallas import tpu_sc as plsc

D_MODEL = 64
SCALE = math.sqrt(D_MODEL)

NUM_CORES = 2
NUM_SUBCORES = 16
NUM_WORKERS = NUM_CORES * NUM_SUBCORES
LANES = 16

BLK = 128


@functools.lru_cache(maxsize=None)
def _build(num_blocks: int, vocab: int):
    blocks_per_w = num_blocks // NUM_WORKERS
    mesh = plsc.VectorSubcoreMesh(
        core_axis_name="c", subcore_axis_name="s",
        num_cores=NUM_CORES, num_subcores=NUM_SUBCORES,
    )

    @functools.partial(
        pl.kernel,
        out_type=jax.ShapeDtypeStruct((num_blocks * BLK, D_MODEL), jnp.float32),
        mesh=mesh,
        scratch_types=[
            pltpu.VMEM((blocks_per_w, BLK), jnp.int32),
            pltpu.VMEM((BLK, D_MODEL), jnp.float32),
            pltpu.SemaphoreType.DMA,
        ],
        compiler_params=pltpu.CompilerParams(use_tc_tiling_on_sc=False),
    )
    def gather_kernel(idx_hbm, table_hbm, out_hbm, idx_v, rows_v, sem):
        wid = lax.axis_index("s") * NUM_CORES + lax.axis_index("c")
        blk0 = wid * blocks_per_w
        pltpu.sync_copy(idx_hbm.at[pl.ds(blk0, blocks_per_w)], idx_v)

        def blk_body(g, carry):
            pltpu.async_copy(table_hbm.at[idx_v.at[g]], rows_v, sem).wait()

            def row_body(i, c):
                for j in range(D_MODEL // LANES):
                    sl = (i, pl.ds(j * LANES, LANES))
                    rows_v[sl] = rows_v[sl] * SCALE
                return c

            lax.fori_loop(0, BLK, row_body, 0)
            pltpu.sync_copy(rows_v, out_hbm.at[pl.ds((blk0 + g) * BLK, BLK)])
            return carry

        lax.fori_loop(0, blocks_per_w, blk_body, 0)

    return gather_kernel


def kernel(x, embed_weight):
    b, s = x.shape
    n = b * s
    idx = x.reshape(n // BLK, BLK).astype(jnp.int32)
    fn = _build(n // BLK, embed_weight.shape[0])
    out = fn(idx, embed_weight)
    return out.reshape(b, s, D_MODEL)


# speedup vs baseline: 1.2133x; 1.2133x over previous
import functools
import math

import jax
import jax.numpy as jnp
from jax import lax
from jax.experimental import pallas as pl
from jax.experimental.pallas import tpu as pltpu
from jax.experimental.pallas import tpu_sc as plsc

D_MODEL = 64
SCALE = math.sqrt(D_MODEL)

NUM_CORES = 2
NUM_SUBCORES = 16
NUM_WORKERS = NUM_CORES * NUM_SUBCORES
LANES = 16

BLK = 128
NBUF = 4
ROW_UNROLL = 8


@functools.lru_cache(maxsize=None)
def _build(num_blocks: int, vocab: int):
    blocks_per_w = num_blocks // NUM_WORKERS
    outer_iters = blocks_per_w // NBUF
    mesh = plsc.VectorSubcoreMesh(
        core_axis_name="c", subcore_axis_name="s",
        num_cores=NUM_CORES, num_subcores=NUM_SUBCORES,
    )

    @functools.partial(
        pl.kernel,
        out_type=jax.ShapeDtypeStruct((num_blocks * BLK, D_MODEL), jnp.float32),
        mesh=mesh,
        scratch_types=[
            pltpu.VMEM((blocks_per_w, BLK), jnp.int32),
            pltpu.VMEM((NBUF, BLK, D_MODEL), jnp.float32),
            pltpu.VMEM((NBUF, BLK, D_MODEL), jnp.float32),
        ] + [pltpu.SemaphoreType.DMA] * (2 * NBUF),
        compiler_params=pltpu.CompilerParams(use_tc_tiling_on_sc=False),
    )
    def gather_kernel(idx_hbm, table_hbm, out_hbm, idx_v, ibuf, obuf, *sems):
        gsems = sems[:NBUF]
        ssems = sems[NBUF:]
        wid = lax.axis_index("s") * NUM_CORES + lax.axis_index("c")
        blk0 = wid * blocks_per_w
        pltpu.sync_copy(idx_hbm.at[pl.ds(blk0, blocks_per_w)], idx_v)

        for b in range(NBUF):
            pltpu.async_copy(table_hbm.at[idx_v.at[b]], ibuf.at[b], gsems[b])

        def outer(t, carry):
            for b in range(NBUF):
                g = t * NBUF + b
                pltpu.make_async_copy(
                    table_hbm.at[idx_v.at[0]], ibuf.at[b], gsems[b]).wait()

                @pl.when(t > 0)
                def _wait_scatter(b=b):
                    pltpu.make_async_copy(
                        obuf.at[b], out_hbm.at[pl.ds(0, BLK)], ssems[b]).wait()

                def scale(i, c, b=b):
                    for r in range(ROW_UNROLL):
                        row = i * ROW_UNROLL + r
                        for j in range(D_MODEL // LANES):
                            sl = pl.ds(j * LANES, LANES)
                            obuf[b, row, sl] = ibuf[b, row, sl] * SCALE
                    return c

                lax.fori_loop(0, BLK // ROW_UNROLL, scale, 0)

                @pl.when(t < outer_iters - 1)
                def _next_gather(b=b, g=g):
                    pltpu.async_copy(
                        table_hbm.at[idx_v.at[g + NBUF]], ibuf.at[b], gsems[b])

                pltpu.async_copy(
                    obuf.at[b], out_hbm.at[pl.ds((blk0 + g) * BLK, BLK)],
                    ssems[b])
            return carry

        lax.fori_loop(0, outer_iters, outer, 0)

        for b in range(NBUF):
            pltpu.make_async_copy(
                obuf.at[b], out_hbm.at[pl.ds(0, BLK)], ssems[b]).wait()

    return gather_kernel


def kernel(x, embed_weight):
    b, s = x.shape
    n = b * s
    idx = x.reshape(n // BLK, BLK).astype(jnp.int32)
    fn = _build(n // BLK, embed_weight.shape[0])
    out = fn(idx, embed_weight)
    return out.reshape(b, s, D_MODEL)
